# baseline (speedup 1.0000x reference)
import functools

import jax
import jax.numpy as jnp
from jax import lax
from jax.experimental import pallas as pl
from jax.experimental.pallas import tpu as pltpu
from jax.experimental.pallas import tpu_sc as plsc

N = 100000
E = 6400000
H = 16

NC = 2
NS = 16
NW = NC * NS

R = 784
Np = R * 128
ZW = Np // NS

CHUNK = 2048
CHUNK_DEG = 10240

_mesh = plsc.VectorSubcoreMesh(core_axis_name="c", subcore_axis_name="s")


NSLOT = 3


def _sc_scatter_body(gather, CHUNK, *refs):
    NCHUNKS = E // CHUNK
    if gather:
        (ei_hbm, tab_hbm, out_hbm,
         is0, is1, is2, id0, id1, id2, va0, va1, va2,
         tab_v, acc_sh, sem_in, sem_out) = refs
        idx_s = [is0, is1, is2]
        SCOFF = E
    else:
        (ei_hbm, out_hbm,
         id0, id1, id2, va0, va1, va2,
         acc_sh, sem_in, sem_out) = refs
        idx_s = None
        SCOFF = 0
    vdt = jnp.float32
    VL = 16
    idx_d = [id0, id1, id2]
    vals = [va0, va1, va2]
    c = lax.axis_index("c")
    s = lax.axis_index("s")
    w = s * NC + c

    ZC = ZW // 4
    for j in range(ZC // VL):
        vals[0][pl.ds(j * VL, VL)] = jnp.zeros((VL,), vdt)
    for q in range(4):
        pltpu.sync_copy(vals[0].at[pl.ds(0, ZC)],
                        acc_sh.at[pl.ds(s * ZW + q * ZC, ZC)])

    if gather:
        pltpu.sync_copy(tab_hbm, tab_v)
    else:
        for b in range(NSLOT):
            for j in range(CHUNK // VL):
                vals[b][pl.ds(j * VL, VL)] = jnp.ones((VL,), vdt)
    plsc.subcore_barrier()

    n_my = (NCHUNKS - w + NW - 1) // NW

    def chunk_off(r):
        return (w + r * NW) * CHUNK

    def start_in(r, b):
        pltpu.async_copy(ei_hbm.at[pl.ds(SCOFF + chunk_off(r), CHUNK)],
                         idx_d[b], sem_in.at[b])
        if gather:
            pltpu.async_copy(ei_hbm.at[pl.ds(chunk_off(r), CHUNK)],
                             idx_s[b], sem_in.at[b])

    def wait_in(b):
        pltpu.make_async_copy(ei_hbm.at[pl.ds(0, CHUNK)],
                              idx_d[b], sem_in.at[b]).wait()
        if gather:
            pltpu.make_async_copy(ei_hbm.at[pl.ds(0, CHUNK)],
                                  idx_s[b], sem_in.at[b]).wait()

    def do_gather(b):
        if gather:
            G = 16
            for blk in range(0, CHUNK // 16, G):
                ivs = [idx_s[b][pl.ds((blk + t) * 16, 16)] for t in range(G)]
                gs = [plsc.load_gather(tab_v, [iv]) for iv in ivs]
                for t in range(G):
                    vals[b][pl.ds((blk + t) * 16, 16)] = gs[t]

    def start_scatter(b):
        pltpu.async_copy(vals[b], acc_sh.at[idx_d[b]],
                         sem_out.at[b], add=True)

    def wait_scatter(b):
        pltpu.make_async_copy(vals[b], acc_sh.at[pl.ds(0, CHUNK)],
                              sem_out.at[b]).wait()

    for b in range(NSLOT):
        start_in(b, b)
    wait_in(0)
    do_gather(0)
    start_scatter(0)
    for r in (1, 2):
        bp = (r - 1) % NSLOT
        b = r % NSLOT
        wait_scatter(bp)
        start_in(r + 2, bp)
        wait_in(b)
        do_gather(b)
        start_scatter(b)

    def group(g, carry):
        for b in range(NSLOT):
            r = g * NSLOT + b

            @pl.when(r < n_my)
            def _():
                bp = (b - 1) % NSLOT
                wait_scatter(bp)

                @pl.when(r + 2 < n_my)
                def _():
                    start_in(r + 2, bp)
                wait_in(b)
                do_gather(b)
                start_scatter(b)
        return carry

    n_groups = (n_my + NSLOT - 1) // NSLOT
    lax.fori_loop(1, n_groups, group, 0)

    for b in range(NSLOT):
        @pl.when((n_my - 1) % NSLOT == b)
        def _():
            wait_scatter(b)

    plsc.subcore_barrier()
    pltpu.sync_copy(acc_sh.at[pl.ds(s * ZW, ZW)],
                    out_hbm.at[c].at[pl.ds(s * ZW, ZW)])


def _sc_scatter(ei, tab):
    gather = tab is not None
    chunk = CHUNK if gather else CHUNK_DEG
    body = functools.partial(_sc_scatter_body, gather, chunk)
    n_idx = 2 * NSLOT if gather else NSLOT
    scratch = (
        [pltpu.VMEM((chunk,), jnp.int32) for _ in range(n_idx)]
        + [pltpu.VMEM((chunk,), jnp.float32) for _ in range(NSLOT)]
        + ([pltpu.VMEM((Np,), jnp.float32)] if gather else [])
        + [
            pltpu.VMEM_SHARED((Np,), jnp.float32),
            pltpu.SemaphoreType.DMA((NSLOT,)),
            pltpu.SemaphoreType.DMA((NSLOT,)),
        ]
    )
    k = pl.kernel(
        body,
        out_type=jax.ShapeDtypeStruct((2, Np), jnp.float32),
        mesh=_mesh,
        scratch_types=scratch,
        compiler_params=pltpu.CompilerParams(needs_layout_passes=False),
    )
    if gather:
        return k(ei, tab)
    return k(ei)


def _tc_deg_stage(deg2, x2):
    def body(d_ref, x_ref, dis_ref, p_ref):
        deg = (d_ref[0] + d_ref[1]).astype(jnp.float32)
        dis = jnp.where(deg > 0,
                        lax.rsqrt(jnp.maximum(deg, 1e-12)),
                        0.0)
        dis_ref[...] = dis
        p_ref[...] = dis * x_ref[...]
    return pl.pallas_call(
        body,
        out_shape=[jax.ShapeDtypeStruct((R, 128), jnp.float32),
                   jax.ShapeDtypeStruct((R, 128), jnp.float32)],
    )(deg2, x2)


def _tc_hidden_stage(q2, dis2, x2, wpack):
    def body(q_ref, dis_ref, x_ref, w_ref, t_ref, r_ref):
        dis = dis_ref[...]
        s1 = -dis * (q_ref[0] + q_ref[1])
        x = x_ref[...]
        t = jnp.zeros_like(x)
        g = jnp.zeros_like(x)
        for j in range(H):
            h = jnp.maximum(w_ref[0, j] * x + w_ref[1, j] * s1 + w_ref[2, j],
                            0.0)
            t = t + w_ref[3, j] * h
            g = g + w_ref[4, j] * h
        t_ref[...] = t + w_ref[5, 0]
        r_ref[...] = dis * g
    return pl.pallas_call(
        body,
        in_specs=[pl.BlockSpec(memory_space=pltpu.MemorySpace.VMEM),
                  pl.BlockSpec(memory_space=pltpu.MemorySpace.VMEM),
                  pl.BlockSpec(memory_space=pltpu.MemorySpace.VMEM),
                  pl.BlockSpec(memory_space=pltpu.MemorySpace.SMEM)],
        out_shape=[jax.ShapeDtypeStruct((R, 128), jnp.float32),
                   jax.ShapeDtypeStruct((R, 128), jnp.float32)],
    )(q2, dis2, x2, wpack)


def _tc_out_stage(t2, dis2, q2):
    def body(t_ref, dis_ref, q_ref, o_ref):
        o_ref[...] = t_ref[...] - dis_ref[...] * (q_ref[0] + q_ref[1])
    return pl.pallas_call(
        body,
        out_shape=jax.ShapeDtypeStruct((R, 128), jnp.float32),
    )(t2, dis2, q2)


def kernel(x, edge_index, W1, b1, W2, b2):
    ei = edge_index.reshape(2 * E)
    x2 = jnp.pad(x[:, 0], (0, Np - N)).reshape(R, 128)

    wpack = jnp.stack([
        W1[0, 0, :], W1[1, 0, :], b1,
        W2[0, :, 0], W2[1, :, 0],
        jnp.broadcast_to(b2, (H,)),
    ])

    deg2 = _sc_scatter(ei, None)
    dis2, p2 = _tc_deg_stage(deg2.reshape(2, R, 128), x2)
    q12 = _sc_scatter(ei, p2.reshape(Np))
    t2, r2 = _tc_hidden_stage(q12.reshape(2, R, 128), dis2, x2, wpack)
    q22 = _sc_scatter(ei, r2.reshape(Np))
    out2 = _tc_out_stage(t2, dis2, q22.reshape(2, R, 128))

    return out2.reshape(Np)[:N].reshape(N, 1)


# speedup vs baseline: 1.0046x; 1.0046x over previous
import functools

import jax
import jax.numpy as jnp
from jax import lax
from jax.experimental import pallas as pl
from jax.experimental.pallas import tpu as pltpu
from jax.experimental.pallas import tpu_sc as plsc

N = 100000
E = 6400000
H = 16

NC = 2
NS = 16
NW = NC * NS

R = 784
Np = R * 128
ZW = Np // NS

CHUNK = 2560
CHUNK_DEG = 10240

_mesh = plsc.VectorSubcoreMesh(core_axis_name="c", subcore_axis_name="s")


NSLOT = 3


def _sc_scatter_body(gather, CHUNK, *refs):
    NCHUNKS = E // CHUNK
    if gather:
        (ei_hbm, tab_hbm, out_hbm,
         is0, is1, is2, id0, id1, id2, va0, va1, va2,
         tab_v, acc_sh, sem_in, sem_out) = refs
        idx_s = [is0, is1, is2]
        SCOFF = E
    else:
        (ei_hbm, out_hbm,
         id0, id1, id2, va0, va1, va2,
         acc_sh, sem_in, sem_out) = refs
        idx_s = None
        SCOFF = 0
    vdt = jnp.float32
    VL = 16
    idx_d = [id0, id1, id2]
    vals = [va0, va1, va2]
    c = lax.axis_index("c")
    s = lax.axis_index("s")
    w = s * NC + c

    ZC = ZW // 4
    for j in range(ZC // VL):
        vals[0][pl.ds(j * VL, VL)] = jnp.zeros((VL,), vdt)
    for q in range(4):
        pltpu.sync_copy(vals[0].at[pl.ds(0, ZC)],
                        acc_sh.at[pl.ds(s * ZW + q * ZC, ZC)])

    if gather:
        pltpu.sync_copy(tab_hbm, tab_v)
    else:
        for b in range(NSLOT):
            for j in range(CHUNK // VL):
                vals[b][pl.ds(j * VL, VL)] = jnp.ones((VL,), vdt)
    plsc.subcore_barrier()

    n_my = (NCHUNKS - w + NW - 1) // NW

    def chunk_off(r):
        return (w + r * NW) * CHUNK

    def start_in(r, b):
        pltpu.async_copy(ei_hbm.at[pl.ds(SCOFF + chunk_off(r), CHUNK)],
                         idx_d[b], sem_in.at[b])
        if gather:
            pltpu.async_copy(ei_hbm.at[pl.ds(chunk_off(r), CHUNK)],
                             idx_s[b], sem_in.at[b])

    def wait_in(b):
        pltpu.make_async_copy(ei_hbm.at[pl.ds(0, CHUNK)],
                              idx_d[b], sem_in.at[b]).wait()
        if gather:
            pltpu.make_async_copy(ei_hbm.at[pl.ds(0, CHUNK)],
                                  idx_s[b], sem_in.at[b]).wait()

    def do_gather(b):
        if gather:
            G = 8
            for blk in range(0, CHUNK // 16, G):
                ivs = [idx_s[b][pl.ds((blk + t) * 16, 16)] for t in range(G)]
                gs = [plsc.load_gather(tab_v, [iv]) for iv in ivs]
                for t in range(G):
                    vals[b][pl.ds((blk + t) * 16, 16)] = gs[t]

    def start_scatter(b):
        pltpu.async_copy(vals[b], acc_sh.at[idx_d[b]],
                         sem_out.at[b], add=True)

    def wait_scatter(b):
        pltpu.make_async_copy(vals[b], acc_sh.at[pl.ds(0, CHUNK)],
                              sem_out.at[b]).wait()

    for b in range(NSLOT):
        start_in(b, b)
    wait_in(0)
    do_gather(0)
    start_scatter(0)
    for r in (1, 2):
        bp = (r - 1) % NSLOT
        b = r % NSLOT
        wait_scatter(bp)
        start_in(r + 2, bp)
        wait_in(b)
        do_gather(b)
        start_scatter(b)

    def group(g, carry):
        for b in range(NSLOT):
            r = g * NSLOT + b

            @pl.when(r < n_my)
            def _():
                bp = (b - 1) % NSLOT
                wait_scatter(bp)

                @pl.when(r + 2 < n_my)
                def _():
                    start_in(r + 2, bp)
                wait_in(b)
                do_gather(b)
                start_scatter(b)
        return carry

    n_groups = (n_my + NSLOT - 1) // NSLOT
    lax.fori_loop(1, n_groups, group, 0)

    for b in range(NSLOT):
        @pl.when((n_my - 1) % NSLOT == b)
        def _():
            wait_scatter(b)

    plsc.subcore_barrier()
    pltpu.sync_copy(acc_sh.at[pl.ds(s * ZW, ZW)],
                    out_hbm.at[c].at[pl.ds(s * ZW, ZW)])


def _sc_scatter(ei, tab):
    gather = tab is not None
    chunk = CHUNK if gather else CHUNK_DEG
    body = functools.partial(_sc_scatter_body, gather, chunk)
    n_idx = 2 * NSLOT if gather else NSLOT
    scratch = (
        [pltpu.VMEM((chunk,), jnp.int32) for _ in range(n_idx)]
        + [pltpu.VMEM((chunk,), jnp.float32) for _ in range(NSLOT)]
        + ([pltpu.VMEM((Np,), jnp.float32)] if gather else [])
        + [
            pltpu.VMEM_SHARED((Np,), jnp.float32),
            pltpu.SemaphoreType.DMA((NSLOT,)),
            pltpu.SemaphoreType.DMA((NSLOT,)),
        ]
    )
    k = pl.kernel(
        body,
        out_type=jax.ShapeDtypeStruct((2, Np), jnp.float32),
        mesh=_mesh,
        scratch_types=scratch,
        compiler_params=pltpu.CompilerParams(needs_layout_passes=False),
    )
    if gather:
        return k(ei, tab)
    return k(ei)


def _tc_deg_stage(deg2, x2):
    def body(d_ref, x_ref, dis_ref, p_ref):
        deg = (d_ref[0] + d_ref[1]).astype(jnp.float32)
        dis = jnp.where(deg > 0,
                        lax.rsqrt(jnp.maximum(deg, 1e-12)),
                        0.0)
        dis_ref[...] = dis
        p_ref[...] = dis * x_ref[...]
    return pl.pallas_call(
        body,
        out_shape=[jax.ShapeDtypeStruct((R, 128), jnp.float32),
                   jax.ShapeDtypeStruct((R, 128), jnp.float32)],
    )(deg2, x2)


def _tc_hidden_stage(q2, dis2, x2, wpack):
    def body(q_ref, dis_ref, x_ref, w_ref, t_ref, r_ref):
        dis = dis_ref[...]
        s1 = -dis * (q_ref[0] + q_ref[1])
        x = x_ref[...]
        t = jnp.zeros_like(x)
        g = jnp.zeros_like(x)
        for j in range(H):
            h = jnp.maximum(w_ref[0, j] * x + w_ref[1, j] * s1 + w_ref[2, j],
                            0.0)
            t = t + w_ref[3, j] * h
            g = g + w_ref[4, j] * h
        t_ref[...] = t + w_ref[5, 0]
        r_ref[...] = dis * g
    return pl.pallas_call(
        body,
        in_specs=[pl.BlockSpec(memory_space=pltpu.MemorySpace.VMEM),
                  pl.BlockSpec(memory_space=pltpu.MemorySpace.VMEM),
                  pl.BlockSpec(memory_space=pltpu.MemorySpace.VMEM),
                  pl.BlockSpec(memory_space=pltpu.MemorySpace.SMEM)],
        out_shape=[jax.ShapeDtypeStruct((R, 128), jnp.float32),
                   jax.ShapeDtypeStruct((R, 128), jnp.float32)],
    )(q2, dis2, x2, wpack)


def _tc_out_stage(t2, dis2, q2):
    def body(t_ref, dis_ref, q_ref, o_ref):
        o_ref[...] = t_ref[...] - dis_ref[...] * (q_ref[0] + q_ref[1])
    return pl.pallas_call(
        body,
        out_shape=jax.ShapeDtypeStruct((R, 128), jnp.float32),
    )(t2, dis2, q2)


def kernel(x, edge_index, W1, b1, W2, b2):
    ei = edge_index.reshape(2 * E)
    x2 = jnp.pad(x[:, 0], (0, Np - N)).reshape(R, 128)

    wpack = jnp.stack([
        W1[0, 0, :], W1[1, 0, :], b1,
        W2[0, :, 0], W2[1, :, 0],
        jnp.broadcast_to(b2, (H,)),
    ])

    deg2 = _sc_scatter(ei, None)
    dis2, p2 = _tc_deg_stage(deg2.reshape(2, R, 128), x2)
    q12 = _sc_scatter(ei, p2.reshape(Np))
    t2, r2 = _tc_hidden_stage(q12.reshape(2, R, 128), dis2, x2, wpack)
    q22 = _sc_scatter(ei, r2.reshape(Np))
    out2 = _tc_out_stage(t2, dis2, q22.reshape(2, R, 128))

    return out2.reshape(Np)[:N].reshape(N, 1)
